# Initial kernel scaffold; baseline (speedup 1.0000x reference)
import jax
import jax.numpy as jnp
from jax.experimental import pallas as pl


def kernel(inputs, embed, W1, b1, W2, b2, W3, b3):
    raise NotImplementedError("write your pallas kernel here")


# speedup vs baseline: 5.4533x; 5.4533x over previous
import functools

import jax
import jax.numpy as jnp
from jax import lax
from jax.experimental import pallas as pl
from jax.experimental.pallas import tpu as pltpu
from jax.experimental.pallas import tpu_sc as plsc

VOCAB = 100000
EMB = 300
HID = 100
OUT = 5
B = 4096
L = 200

DPAD = 128
NLANE = 16
NSEG = DPAD // NLANE
NW = 32
ROWS_PER_W = B // NW
C0, C1 = 104, 96


def _mm_body(x_ref, w_ref, o_ref):
    o_ref[...] = jnp.dot(x_ref[...], w_ref[...],
                         preferred_element_type=jnp.float32)


def _table_matmul(embed, w1pad):
    vb = 2000
    return pl.pallas_call(
        _mm_body,
        grid=(VOCAB // vb,),
        in_specs=[
            pl.BlockSpec((vb, EMB), lambda i: (i, 0)),
            pl.BlockSpec((EMB, DPAD), lambda i: (0, 0)),
        ],
        out_specs=pl.BlockSpec((vb, DPAD), lambda i: (i, 0)),
        out_shape=jax.ShapeDtypeStruct((VOCAB, DPAD), jnp.float32),
    )(embed, w1pad)


def _sc_body(idx_hbm, tab_hbm, out_hbm, idx_v, buf0, buf1, out_v, sem0, sem1):
    wid = lax.axis_index("s") * 2 + lax.axis_index("c")
    base = wid * ROWS_PER_W
    pltpu.sync_copy(idx_hbm.at[pl.ds(base * L, ROWS_PER_W * L)], idx_v)

    def fire(r, buf, sem):
        o0 = pl.multiple_of(r * L, 8)
        o1 = pl.multiple_of(r * L + C0, 8)
        pltpu.async_copy(tab_hbm.at[idx_v.at[pl.ds(o0, C0)]],
                         buf.at[pl.ds(0, C0)], sem)
        pltpu.async_copy(tab_hbm.at[idx_v.at[pl.ds(o1, C1)]],
                         buf.at[pl.ds(C0, C1)], sem)

    def wait(r, buf, sem):
        o0 = pl.multiple_of(r * L, 8)
        o1 = pl.multiple_of(r * L + C0, 8)
        pltpu.make_async_copy(tab_hbm.at[idx_v.at[pl.ds(o0, C0)]],
                              buf.at[pl.ds(0, C0)], sem).wait()
        pltpu.make_async_copy(tab_hbm.at[idx_v.at[pl.ds(o1, C1)]],
                              buf.at[pl.ds(C0, C1)], sem).wait()

    def accum(r, buf):
        def body(j, accs):
            return tuple(accs[c] + buf[j, pl.ds(c * NLANE, NLANE)]
                         for c in range(NSEG))
        init = tuple(jnp.zeros((NLANE,), jnp.float32) for _ in range(NSEG))
        accs = lax.fori_loop(0, L, body, init, unroll=8)
        for c in range(NSEG):
            out_v[r, pl.ds(c * NLANE, NLANE)] = accs[c]

    fire(0, buf0, sem0)

    def outer(i, carry):
        r0 = 2 * i
        fire(r0 + 1, buf1, sem1)
        wait(r0, buf0, sem0)
        accum(r0, buf0)

        @pl.when(i + 1 < ROWS_PER_W // 2)
        def _():
            fire(r0 + 2, buf0, sem0)

        wait(r0 + 1, buf1, sem1)
        accum(r0 + 1, buf1)
        return carry

    lax.fori_loop(0, ROWS_PER_W // 2, outer, 0)
    pltpu.sync_copy(out_v, out_hbm.at[pl.ds(base, ROWS_PER_W)])


def _sc_gather_sum(idx_flat, table):
    f = functools.partial(
        pl.kernel,
        out_type=jax.ShapeDtypeStruct((B, DPAD), jnp.float32),
        mesh=plsc.VectorSubcoreMesh(core_axis_name="c", subcore_axis_name="s"),
        scratch_types=[
            pltpu.VMEM((ROWS_PER_W * L,), jnp.int32),
            pltpu.VMEM((L, DPAD), jnp.float32),
            pltpu.VMEM((L, DPAD), jnp.float32),
            pltpu.VMEM((ROWS_PER_W, DPAD), jnp.float32),
            pltpu.SemaphoreType.DMA,
            pltpu.SemaphoreType.DMA,
        ],
    )(_sc_body)
    return f(idx_flat, table)


def _mlp_body(s_ref, b1_ref, w2_ref, b2_ref, w3_ref, b3_ref, o_ref):
    h = jnp.tanh(s_ref[...] + b1_ref[...])
    h = jnp.tanh(jnp.dot(h, w2_ref[...], preferred_element_type=jnp.float32)
                 + b2_ref[...])
    o_ref[...] = (jnp.dot(h, w3_ref[...], preferred_element_type=jnp.float32)
                  + b3_ref[...])


def _mlp(s, b1p, w2p, b2p, w3p, b3p):
    bb = 512
    return pl.pallas_call(
        _mlp_body,
        grid=(B // bb,),
        in_specs=[
            pl.BlockSpec((bb, DPAD), lambda i: (i, 0)),
            pl.BlockSpec((1, DPAD), lambda i: (0, 0)),
            pl.BlockSpec((DPAD, DPAD), lambda i: (0, 0)),
            pl.BlockSpec((1, DPAD), lambda i: (0, 0)),
            pl.BlockSpec((DPAD, OUT), lambda i: (0, 0)),
            pl.BlockSpec((1, OUT), lambda i: (0, 0)),
        ],
        out_specs=pl.BlockSpec((bb, OUT), lambda i: (i, 0)),
        out_shape=jax.ShapeDtypeStruct((B, OUT), jnp.float32),
    )(s, b1p, w2p, b2p, w3p, b3p)


def kernel(inputs, embed, W1, b1, W2, b2, W3, b3):
    w1p = jnp.zeros((EMB, DPAD), jnp.float32).at[:, :HID].set(W1)
    table = _table_matmul(embed, w1p)
    s = _sc_gather_sum(inputs.reshape(-1), table)
    b1p = jnp.zeros((1, DPAD), jnp.float32).at[0, :HID].set(b1)
    w2p = jnp.zeros((DPAD, DPAD), jnp.float32).at[:HID, :HID].set(W2)
    b2p = jnp.zeros((1, DPAD), jnp.float32).at[0, :HID].set(b2)
    w3p = jnp.zeros((DPAD, OUT), jnp.float32).at[:HID, :].set(W3)
    b3p = b3.reshape(1, OUT)
    return _mlp(s, b1p, w2p, b2p, w3p, b3p)
